# baseline (speedup 1.0000x reference)
import functools

import jax
import jax.numpy as jnp
from jax.experimental import pallas as pl
from jax.experimental.pallas import tpu as pltpu

D_MODEL = 768
FF = 3072
N_EXPERT = 8
T = 2048
NPAIR = 2 * T

BM = 128
NPAD = NPAIR + N_EXPERT * BM
NB = NPAD // BM
CHUNK = 512
NCH = NPAD // CHUNK
RANK_BLK = 128
N_RANK_BLK = NPAIR // RANK_BLK


def _dot_t(a, b, precision=None):
    return jax.lax.dot_general(a, b, (((1,), (1,)), ((), ())),
                               preferred_element_type=jnp.float32,
                               precision=precision)


def _router_body(s_ref, o_tok, o_p, o_dest, o_blk, o_nv,
                 dest_s, a2_s, oh_s):
    c = pl.program_id(0)

    @pl.when(c == 0)
    def _route():
        scores = s_ref[...]
        m1 = jnp.max(scores, axis=-1, keepdims=True)
        col = jax.lax.broadcasted_iota(jnp.int32, scores.shape, 1)
        col1 = jnp.min(jnp.where(scores == m1, col, N_EXPERT), axis=-1,
                       keepdims=True)
        mask1 = (col == col1).astype(jnp.float32)
        scores2 = jnp.where(col == col1, -jnp.inf, scores)
        m2 = jnp.max(scores2, axis=-1, keepdims=True)
        col2 = jnp.min(jnp.where(scores2 == m2, col, N_EXPERT), axis=-1,
                       keepdims=True)
        mask2 = (col == col2).astype(jnp.float32)
        p1 = 1.0 / (1.0 + jnp.exp(m2 - m1))
        p2 = 1.0 - p1

        oh_s[...] = jnp.concatenate([mask1, mask2], axis=0)

        r = jax.lax.broadcasted_iota(jnp.int32, (RANK_BLK, RANK_BLK), 0)
        cl = jax.lax.broadcasted_iota(jnp.int32, (RANK_BLK, RANK_BLK), 1)
        tri = (cl < r).astype(jnp.float32)

        def _cs(i, carry):
            chunk = oh_s[pl.ds(i * RANK_BLK, RANK_BLK), :]
            excl = jnp.dot(tri, chunk, preferred_element_type=jnp.float32)
            a2_s[pl.ds(i * RANK_BLK, RANK_BLK), 2:2 + N_EXPERT] = excl + carry
            return carry + jnp.sum(chunk, axis=0, keepdims=True)

        counts = jax.lax.fori_loop(0, N_RANK_BLK, _cs,
                                   jnp.zeros((1, N_EXPERT), jnp.float32))

        padded = jnp.ceil(counts / BM) * BM
        er = jax.lax.broadcasted_iota(jnp.int32, (N_EXPERT, N_EXPERT), 0)
        ec = jax.lax.broadcasted_iota(jnp.int32, (N_EXPERT, N_EXPERT), 1)
        etri = (er < ec).astype(jnp.float32)
        pad_off = jnp.dot(padded, etri,
                          preferred_element_type=jnp.float32)

        rank = a2_s[:, 2:2 + N_EXPERT]
        dest = jnp.sum((rank + pad_off) * oh_s[...], axis=-1, keepdims=True)
        dest_s[...] = dest
        o_dest[...] = dest.astype(jnp.int32)

        pr = jax.lax.broadcasted_iota(jnp.int32, (NPAIR, 1), 0)
        tok = jnp.where(pr < T, pr, pr - T).astype(jnp.float32)
        prob = jnp.concatenate([p1, p2], axis=0)
        a2_s[:, 0:1] = tok
        a2_s[:, 1:2] = prob

        pad_end = pad_off + padded
        total = jnp.sum(padded).astype(jnp.float32)
        bstart = (jax.lax.broadcasted_iota(jnp.int32, (128, 1), 0)
                  .astype(jnp.float32) * BM)
        raw = jnp.sum((pad_end <= bstart).astype(jnp.int32), axis=-1,
                      keepdims=True)
        last_used = jnp.sum((pad_end < total).astype(jnp.int32))
        o_blk[...] = jnp.where(bstart < total, raw, last_used)
        o_nv[...] = jnp.full((8, 1), (total / BM), jnp.float32).astype(jnp.int32)

    jcol = (jax.lax.broadcasted_iota(jnp.int32, (1, CHUNK), 1)
            .astype(jnp.float32) + jnp.float32(c * CHUNK))
    m = (dest_s[...] == jcol).astype(jnp.float32)
    res = jax.lax.dot_general(m, a2_s[:, 0:2], (((0,), (0,)), ((), ())),
                              preferred_element_type=jnp.float32,
                              precision=jax.lax.Precision.HIGHEST)
    o_tok[...] = res[:, 0:1].astype(jnp.int32)
    o_p[...] = res[:, 1:2]


def _ffn_body(blk_ref, nv_ref, tok_ref, x_ref, w1_ref, w2_ref, p_ref,
              out_ref, xg_s):
    b = pl.program_id(0)

    @pl.when(b < nv_ref[0])
    def _work():
        def _gather(i, _):
            xg_s[i, :] = x_ref[tok_ref[b * BM + i], :]
            return 0
        jax.lax.fori_loop(0, BM, _gather, 0)
        xw1 = _dot_t(xg_s[...], w1_ref[0])
        h = xw1 * jax.nn.sigmoid(xw1)
        out_ref[...] = _dot_t(h, w2_ref[0]) * p_ref[...]


def _combine_body(dest_ref, po_ref, y_ref):
    tb = pl.program_id(0)

    def _comb(i, _):
        t = tb * 256 + i
        y_ref[i, :] = po_ref[dest_ref[t], :] + po_ref[dest_ref[T + t], :]
        return 0
    jax.lax.fori_loop(0, 256, _comb, 0)


@functools.partial(jax.jit, static_argnames=())
def kernel(x, Wg, W1, W2):
    B, Tn, C = x.shape
    x_flat = x.reshape(Tn, C)

    scores = x_flat @ Wg.T

    tok_s, p_s, dest, blk, nv = pl.pallas_call(
        _router_body,
        grid=(NCH,),
        in_specs=[
            pl.BlockSpec((T, N_EXPERT), lambda c: (0, 0)),
        ],
        out_specs=[
            pl.BlockSpec((CHUNK, 1), lambda c: (c, 0)),
            pl.BlockSpec((CHUNK, 1), lambda c: (c, 0)),
            pl.BlockSpec((NPAIR, 1), lambda c: (0, 0)),
            pl.BlockSpec((128, 1), lambda c: (0, 0)),
            pl.BlockSpec((8, 1), lambda c: (0, 0)),
        ],
        out_shape=[
            jax.ShapeDtypeStruct((NPAD, 1), jnp.int32),
            jax.ShapeDtypeStruct((NPAD, 1), jnp.float32),
            jax.ShapeDtypeStruct((NPAIR, 1), jnp.int32),
            jax.ShapeDtypeStruct((128, 1), jnp.int32),
            jax.ShapeDtypeStruct((8, 1), jnp.int32),
        ],
        scratch_shapes=[
            pltpu.VMEM((NPAIR, 1), jnp.float32),
            pltpu.VMEM((NPAIR, 2 + N_EXPERT), jnp.float32),
            pltpu.VMEM((NPAIR, N_EXPERT), jnp.float32),
        ],
    )(scores)

    pair_out = pl.pallas_call(
        _ffn_body,
        grid_spec=pltpu.PrefetchScalarGridSpec(
            num_scalar_prefetch=3,
            grid=(NB,),
            in_specs=[
                pl.BlockSpec((T, C), lambda b, blk, nv, tok: (0, 0)),
                pl.BlockSpec((1, FF, C), lambda b, blk, nv, tok: (blk[b], 0, 0)),
                pl.BlockSpec((1, C, FF), lambda b, blk, nv, tok: (blk[b], 0, 0)),
                pl.BlockSpec((BM, 1), lambda b, blk, nv, tok: (b, 0)),
            ],
            out_specs=pl.BlockSpec((BM, C), lambda b, blk, nv, tok: (b, 0)),
            scratch_shapes=[pltpu.VMEM((BM, C), jnp.float32)],
        ),
        out_shape=jax.ShapeDtypeStruct((NPAD, C), jnp.float32),
    )(blk.reshape(128), nv.reshape(8)[:1], tok_s.reshape(NPAD),
      x_flat, W1, W2, p_s)

    y = pl.pallas_call(
        _combine_body,
        grid_spec=pltpu.PrefetchScalarGridSpec(
            num_scalar_prefetch=1,
            grid=(T // 256,),
            in_specs=[pl.BlockSpec((NPAD, C), lambda tb, dest: (0, 0))],
            out_specs=pl.BlockSpec((256, C), lambda tb, dest: (tb, 0)),
        ),
        out_shape=jax.ShapeDtypeStruct((T, C), jnp.float32),
    )(dest.reshape(NPAIR), pair_out)

    return y.reshape(B, Tn, C)


# speedup vs baseline: 1.1276x; 1.1276x over previous
import functools

import jax
import jax.numpy as jnp
from jax.experimental import pallas as pl
from jax.experimental.pallas import tpu as pltpu

D_MODEL = 768
FF = 3072
N_EXPERT = 8
T = 2048
NPAIR = 2 * T

BM = 128
NPAD = NPAIR + N_EXPERT * BM
NB = NPAD // BM
RANK_BLK = 128
N_RANK_BLK = NPAIR // RANK_BLK


def _dot_t(a, b, precision=None):
    return jax.lax.dot_general(a, b, (((1,), (1,)), ((), ())),
                               preferred_element_type=jnp.float32,
                               precision=precision)


def _router_body(s_ref, o_dest, o_prob, o_blk, o_nv, rank_s, oh_s):
    scores = s_ref[...]
    m1 = jnp.max(scores, axis=-1, keepdims=True)
    col = jax.lax.broadcasted_iota(jnp.int32, scores.shape, 1)
    col1 = jnp.min(jnp.where(scores == m1, col, N_EXPERT), axis=-1,
                   keepdims=True)
    mask1 = (col == col1).astype(jnp.float32)
    scores2 = jnp.where(col == col1, -jnp.inf, scores)
    m2 = jnp.max(scores2, axis=-1, keepdims=True)
    col2 = jnp.min(jnp.where(scores2 == m2, col, N_EXPERT), axis=-1,
                   keepdims=True)
    mask2 = (col == col2).astype(jnp.float32)
    p1 = 1.0 / (1.0 + jnp.exp(m2 - m1))
    p2 = 1.0 - p1
    o_prob[...] = jnp.concatenate([p1, p2], axis=0)

    oh_s[...] = jnp.concatenate([mask1, mask2], axis=0)

    r = jax.lax.broadcasted_iota(jnp.int32, (RANK_BLK, RANK_BLK), 0)
    cl = jax.lax.broadcasted_iota(jnp.int32, (RANK_BLK, RANK_BLK), 1)
    tri = (cl < r).astype(jnp.float32)

    def _cs(i, carry):
        chunk = oh_s[pl.ds(i * RANK_BLK, RANK_BLK), :]
        excl = jnp.dot(tri, chunk, preferred_element_type=jnp.float32)
        rank_s[pl.ds(i * RANK_BLK, RANK_BLK), :] = excl + carry
        return carry + jnp.sum(chunk, axis=0, keepdims=True)

    counts = jax.lax.fori_loop(0, N_RANK_BLK, _cs,
                               jnp.zeros((1, N_EXPERT), jnp.float32))

    padded = jnp.ceil(counts / BM) * BM
    er = jax.lax.broadcasted_iota(jnp.int32, (N_EXPERT, N_EXPERT), 0)
    ec = jax.lax.broadcasted_iota(jnp.int32, (N_EXPERT, N_EXPERT), 1)
    etri = (er < ec).astype(jnp.float32)
    pad_off = jnp.dot(padded, etri,
                      preferred_element_type=jnp.float32)

    dest = jnp.sum((rank_s[...] + pad_off) * oh_s[...], axis=-1,
                   keepdims=True)
    o_dest[...] = dest.astype(jnp.int32)

    pad_end = pad_off + padded
    total = jnp.sum(padded).astype(jnp.float32)
    bstart = (jax.lax.broadcasted_iota(jnp.int32, (128, 1), 0)
              .astype(jnp.float32) * BM)
    raw = jnp.sum((pad_end <= bstart).astype(jnp.int32), axis=-1,
                  keepdims=True)
    last_used = jnp.sum((pad_end < total).astype(jnp.int32))
    o_blk[...] = jnp.where(bstart < total, raw, last_used)
    o_nv[...] = jnp.full((8, 1), (total / BM), jnp.float32).astype(jnp.int32)


def _invert_body(dest_ref, tok_ref):
    def _inv(p, _):
        d = dest_ref[p]
        tok_ref[d] = p - jnp.where(p >= T, T, 0)
        return 0
    jax.lax.fori_loop(0, NPAIR, _inv, 0)


def _ffn_body(blk_ref, nv_ref, tok_ref, x_ref, w1_ref, w2_ref,
              out_ref, xg_s):
    b = pl.program_id(0)

    @pl.when(b < nv_ref[0])
    def _work():
        def _gather(i, _):
            idx = tok_ref[b * BM + i]
            idx = jnp.minimum(jnp.maximum(idx, 0), T - 1)
            xg_s[i, :] = x_ref[idx, :]
            return 0
        jax.lax.fori_loop(0, BM, _gather, 0)
        xw1 = _dot_t(xg_s[...], w1_ref[0])
        h = xw1 * jax.nn.sigmoid(xw1)
        out_ref[...] = _dot_t(h, w2_ref[0])


def _combine_body(dest_ref, po_ref, p_ref, y_ref):
    tb = pl.program_id(0)

    def _comb(i, _):
        t = tb * 256 + i
        d1 = dest_ref[t]
        d2 = dest_ref[T + t]
        y_ref[i, :] = (p_ref[t] * po_ref[d1, :]
                       + p_ref[T + t] * po_ref[d2, :])
        return 0
    jax.lax.fori_loop(0, 256, _comb, 0)


@functools.partial(jax.jit, static_argnames=())
def kernel(x, Wg, W1, W2):
    B, Tn, C = x.shape
    x_flat = x.reshape(Tn, C)

    scores = x_flat @ Wg.T

    dest, prob, blk, nv = pl.pallas_call(
        _router_body,
        in_specs=[pl.BlockSpec((T, N_EXPERT), lambda: (0, 0))],
        out_specs=[
            pl.BlockSpec((NPAIR, 1), lambda: (0, 0)),
            pl.BlockSpec((NPAIR, 1), lambda: (0, 0)),
            pl.BlockSpec((128, 1), lambda: (0, 0)),
            pl.BlockSpec((8, 1), lambda: (0, 0)),
        ],
        out_shape=[
            jax.ShapeDtypeStruct((NPAIR, 1), jnp.int32),
            jax.ShapeDtypeStruct((NPAIR, 1), jnp.float32),
            jax.ShapeDtypeStruct((128, 1), jnp.int32),
            jax.ShapeDtypeStruct((8, 1), jnp.int32),
        ],
        scratch_shapes=[
            pltpu.VMEM((NPAIR, N_EXPERT), jnp.float32),
            pltpu.VMEM((NPAIR, N_EXPERT), jnp.float32),
        ],
    )(scores)

    dest_flat = dest.reshape(NPAIR)

    tok_s = pl.pallas_call(
        _invert_body,
        in_specs=[pl.BlockSpec(memory_space=pltpu.SMEM)],
        out_specs=pl.BlockSpec(memory_space=pltpu.SMEM),
        out_shape=jax.ShapeDtypeStruct((NPAD,), jnp.int32),
    )(dest_flat)

    pair_out = pl.pallas_call(
        _ffn_body,
        grid_spec=pltpu.PrefetchScalarGridSpec(
            num_scalar_prefetch=3,
            grid=(NB,),
            in_specs=[
                pl.BlockSpec((T, C), lambda b, blk, nv, tok: (0, 0)),
                pl.BlockSpec((1, FF, C), lambda b, blk, nv, tok: (blk[b], 0, 0)),
                pl.BlockSpec((1, C, FF), lambda b, blk, nv, tok: (blk[b], 0, 0)),
            ],
            out_specs=pl.BlockSpec((BM, C), lambda b, blk, nv, tok: (b, 0)),
            scratch_shapes=[pltpu.VMEM((BM, C), jnp.float32)],
        ),
        out_shape=jax.ShapeDtypeStruct((NPAD, C), jnp.float32),
    )(blk.reshape(128), nv.reshape(8)[:1], tok_s, x_flat, W1, W2)

    y = pl.pallas_call(
        _combine_body,
        grid_spec=pltpu.PrefetchScalarGridSpec(
            num_scalar_prefetch=1,
            grid=(T // 256,),
            in_specs=[
                pl.BlockSpec((NPAD, C), lambda tb, dest: (0, 0)),
                pl.BlockSpec(memory_space=pltpu.SMEM),
            ],
            out_specs=pl.BlockSpec((256, C), lambda tb, dest: (tb, 0)),
        ),
        out_shape=jax.ShapeDtypeStruct((T, C), jnp.float32),
    )(dest_flat, pair_out, prob.reshape(NPAIR))

    return y.reshape(B, Tn, C)


# speedup vs baseline: 1.1461x; 1.0164x over previous
import functools

import jax
import jax.numpy as jnp
from jax.experimental import pallas as pl
from jax.experimental.pallas import tpu as pltpu
from jax.experimental.pallas import tpu_sc as plsc

D_MODEL = 768
FF = 3072
N_EXPERT = 8
T = 2048
NPAIR = 2 * T

BM = 128
NPAD = NPAIR + N_EXPERT * BM
NB = NPAD // BM
RANK_BLK = 128
N_RANK_BLK = NPAIR // RANK_BLK

SC_NC = 2
SC_NS = 16
SC_NW = SC_NC * SC_NS
SC_ROWS = NPAD // SC_NW
SC_CHUNK = 80
SC_NCHUNK = SC_ROWS // SC_CHUNK


def _dot_t(a, b, precision=None):
    return jax.lax.dot_general(a, b, (((1,), (1,)), ((), ())),
                               preferred_element_type=jnp.float32,
                               precision=precision)


def _router_body(s_ref, o_dest, o_prob, o_blk, o_nv, rank_s, oh_s):
    scores = s_ref[...]
    m1 = jnp.max(scores, axis=-1, keepdims=True)
    col = jax.lax.broadcasted_iota(jnp.int32, scores.shape, 1)
    col1 = jnp.min(jnp.where(scores == m1, col, N_EXPERT), axis=-1,
                   keepdims=True)
    mask1 = (col == col1).astype(jnp.float32)
    scores2 = jnp.where(col == col1, -jnp.inf, scores)
    m2 = jnp.max(scores2, axis=-1, keepdims=True)
    col2 = jnp.min(jnp.where(scores2 == m2, col, N_EXPERT), axis=-1,
                   keepdims=True)
    mask2 = (col == col2).astype(jnp.float32)
    p1 = 1.0 / (1.0 + jnp.exp(m2 - m1))
    p2 = 1.0 - p1
    o_prob[...] = jnp.concatenate([p1, p2], axis=0)

    oh_s[...] = jnp.concatenate([mask1, mask2], axis=0)

    r = jax.lax.broadcasted_iota(jnp.int32, (RANK_BLK, RANK_BLK), 0)
    cl = jax.lax.broadcasted_iota(jnp.int32, (RANK_BLK, RANK_BLK), 1)
    tri = (cl < r).astype(jnp.float32)

    def _cs(i, carry):
        chunk = oh_s[pl.ds(i * RANK_BLK, RANK_BLK), :]
        excl = jnp.dot(tri, chunk, preferred_element_type=jnp.float32)
        rank_s[pl.ds(i * RANK_BLK, RANK_BLK), :] = excl + carry
        return carry + jnp.sum(chunk, axis=0, keepdims=True)

    counts = jax.lax.fori_loop(0, N_RANK_BLK, _cs,
                               jnp.zeros((1, N_EXPERT), jnp.float32))

    padded = jnp.ceil(counts / BM) * BM
    er = jax.lax.broadcasted_iota(jnp.int32, (N_EXPERT, N_EXPERT), 0)
    ec = jax.lax.broadcasted_iota(jnp.int32, (N_EXPERT, N_EXPERT), 1)
    etri = (er < ec).astype(jnp.float32)
    pad_off = jnp.dot(padded, etri,
                      preferred_element_type=jnp.float32)

    dest = jnp.sum((rank_s[...] + pad_off) * oh_s[...], axis=-1,
                   keepdims=True)
    o_dest[...] = dest.astype(jnp.int32)

    pad_end = pad_off + padded
    total = jnp.sum(padded).astype(jnp.float32)
    bstart = (jax.lax.broadcasted_iota(jnp.int32, (128, 1), 0)
              .astype(jnp.float32) * BM)
    raw = jnp.sum((pad_end <= bstart).astype(jnp.int32), axis=-1,
                  keepdims=True)
    last_used = jnp.sum((pad_end < total).astype(jnp.int32))
    o_blk[...] = jnp.where(bstart < total, raw, last_used)
    o_nv[...] = jnp.full((8, 1), (total / BM), jnp.float32).astype(jnp.int32)


def _invert_body(dest_ref, tok_ref):
    def _inv(p, _):
        d = dest_ref[p]
        tok_ref[d] = p - jnp.where(p >= T, T, 0)
        return 0
    jax.lax.fori_loop(0, NPAIR, _inv, 0, unroll=8)


def _sc_gather_body(x_hbm, idx_hbm, out_hbm, idx_v, rows_v, sem):
    wid = jax.lax.axis_index("s") * SC_NC + jax.lax.axis_index("c")
    base = wid * SC_ROWS
    for ch in range(SC_NCHUNK):
        off = base + ch * SC_CHUNK
        pltpu.sync_copy(idx_hbm.at[pl.ds(off, SC_CHUNK)], idx_v)
        for j in range(SC_CHUNK // 16):
            v = idx_v[pl.ds(j * 16, 16)]
            idx_v[pl.ds(j * 16, 16)] = jnp.minimum(jnp.maximum(v, 0), T - 1)
        pltpu.async_copy(x_hbm.at[idx_v], rows_v, sem).wait()
        pltpu.sync_copy(rows_v, out_hbm.at[pl.ds(off, SC_CHUNK)])


def _ffn_body(blk_ref, nv_ref, xg_ref, w1_ref, w2_ref, out_ref):
    b = pl.program_id(0)

    @pl.when(b < nv_ref[0])
    def _work():
        xw1 = _dot_t(xg_ref[...], w1_ref[0])
        h = xw1 * jax.nn.sigmoid(xw1)
        out_ref[...] = _dot_t(h, w2_ref[0])


def _combine_body(dest_ref, po_ref, p_ref, y_ref):
    tb = pl.program_id(0)

    def _comb(i, _):
        t = tb * 256 + i
        d1 = dest_ref[t]
        d2 = dest_ref[T + t]
        y_ref[i, :] = (p_ref[t] * po_ref[d1, :]
                       + p_ref[T + t] * po_ref[d2, :])
        return 0
    jax.lax.fori_loop(0, 256, _comb, 0)


@functools.partial(jax.jit, static_argnames=())
def kernel(x, Wg, W1, W2):
    B, Tn, C = x.shape
    x_flat = x.reshape(Tn, C)

    scores = x_flat @ Wg.T

    dest, prob, blk, nv = pl.pallas_call(
        _router_body,
        in_specs=[pl.BlockSpec((T, N_EXPERT), lambda: (0, 0))],
        out_specs=[
            pl.BlockSpec((NPAIR, 1), lambda: (0, 0)),
            pl.BlockSpec((NPAIR, 1), lambda: (0, 0)),
            pl.BlockSpec((128, 1), lambda: (0, 0)),
            pl.BlockSpec((8, 1), lambda: (0, 0)),
        ],
        out_shape=[
            jax.ShapeDtypeStruct((NPAIR, 1), jnp.int32),
            jax.ShapeDtypeStruct((NPAIR, 1), jnp.float32),
            jax.ShapeDtypeStruct((128, 1), jnp.int32),
            jax.ShapeDtypeStruct((8, 1), jnp.int32),
        ],
        scratch_shapes=[
            pltpu.VMEM((NPAIR, N_EXPERT), jnp.float32),
            pltpu.VMEM((NPAIR, N_EXPERT), jnp.float32),
        ],
    )(scores)

    dest_flat = dest.reshape(NPAIR)

    tok_s = pl.pallas_call(
        _invert_body,
        in_specs=[pl.BlockSpec(memory_space=pltpu.SMEM)],
        out_specs=pl.BlockSpec(memory_space=pltpu.SMEM),
        out_shape=jax.ShapeDtypeStruct((NPAD,), jnp.int32),
    )(dest_flat)

    xg = pl.kernel(
        _sc_gather_body,
        mesh=plsc.VectorSubcoreMesh(core_axis_name="c", subcore_axis_name="s"),
        out_type=jax.ShapeDtypeStruct((NPAD, C), jnp.float32),
        scratch_types=[
            pltpu.VMEM((SC_CHUNK,), jnp.int32),
            pltpu.VMEM((SC_CHUNK, C), jnp.float32),
            pltpu.SemaphoreType.DMA,
        ],
    )(x_flat, tok_s)

    pair_out = pl.pallas_call(
        _ffn_body,
        grid_spec=pltpu.PrefetchScalarGridSpec(
            num_scalar_prefetch=2,
            grid=(NB,),
            in_specs=[
                pl.BlockSpec((BM, C), lambda b, blk, nv: (b, 0)),
                pl.BlockSpec((1, FF, C), lambda b, blk, nv: (blk[b], 0, 0)),
                pl.BlockSpec((1, C, FF), lambda b, blk, nv: (blk[b], 0, 0)),
            ],
            out_specs=pl.BlockSpec((BM, C), lambda b, blk, nv: (b, 0)),
        ),
        out_shape=jax.ShapeDtypeStruct((NPAD, C), jnp.float32),
    )(blk.reshape(128), nv.reshape(8)[:1], xg, W1, W2)

    y = pl.pallas_call(
        _combine_body,
        grid_spec=pltpu.PrefetchScalarGridSpec(
            num_scalar_prefetch=1,
            grid=(T // 256,),
            in_specs=[
                pl.BlockSpec((NPAD, C), lambda tb, dest: (0, 0)),
                pl.BlockSpec(memory_space=pltpu.SMEM),
            ],
            out_specs=pl.BlockSpec((256, C), lambda tb, dest: (tb, 0)),
        ),
        out_shape=jax.ShapeDtypeStruct((T, C), jnp.float32),
    )(dest_flat, pair_out, prob.reshape(NPAIR))

    return y.reshape(B, Tn, C)


# speedup vs baseline: 1.2672x; 1.1057x over previous
import functools

import jax
import jax.numpy as jnp
from jax.experimental import pallas as pl
from jax.experimental.pallas import tpu as pltpu
from jax.experimental.pallas import tpu_sc as plsc

D_MODEL = 768
FF = 3072
N_EXPERT = 8
T = 2048
NPAIR = 2 * T

BM = 256
NPAD = NPAIR + N_EXPERT * BM
NB = NPAD // BM
RANK_BLK = 128
N_RANK_BLK = NPAIR // RANK_BLK

SC_NC = 2
SC_NS = 16
SC_NW = SC_NC * SC_NS
SC_ROWS = NPAD // SC_NW
SC_CHUNK = 96
SC_NCHUNK = SC_ROWS // SC_CHUNK


def _dot_t(a, b, precision=None):
    return jax.lax.dot_general(a, b, (((1,), (1,)), ((), ())),
                               preferred_element_type=jnp.float32,
                               precision=precision)


def _router_body(s_ref, o_dest, o_prob, o_blk, o_nv, rank_s, oh_s):
    scores = s_ref[...]
    m1 = jnp.max(scores, axis=-1, keepdims=True)
    col = jax.lax.broadcasted_iota(jnp.int32, scores.shape, 1)
    col1 = jnp.min(jnp.where(scores == m1, col, N_EXPERT), axis=-1,
                   keepdims=True)
    mask1 = (col == col1).astype(jnp.float32)
    scores2 = jnp.where(col == col1, -jnp.inf, scores)
    m2 = jnp.max(scores2, axis=-1, keepdims=True)
    col2 = jnp.min(jnp.where(scores2 == m2, col, N_EXPERT), axis=-1,
                   keepdims=True)
    mask2 = (col == col2).astype(jnp.float32)
    p1 = 1.0 / (1.0 + jnp.exp(m2 - m1))
    p2 = 1.0 - p1
    o_prob[...] = jnp.concatenate([p1, p2], axis=0)

    oh_s[...] = jnp.concatenate([mask1, mask2], axis=0)

    r = jax.lax.broadcasted_iota(jnp.int32, (RANK_BLK, RANK_BLK), 0)
    cl = jax.lax.broadcasted_iota(jnp.int32, (RANK_BLK, RANK_BLK), 1)
    tri = (cl < r).astype(jnp.float32)

    def _cs(i, carry):
        chunk = oh_s[pl.ds(i * RANK_BLK, RANK_BLK), :]
        excl = jnp.dot(tri, chunk, preferred_element_type=jnp.float32)
        rank_s[pl.ds(i * RANK_BLK, RANK_BLK), :] = excl + carry
        return carry + jnp.sum(chunk, axis=0, keepdims=True)

    counts = jax.lax.fori_loop(0, N_RANK_BLK, _cs,
                               jnp.zeros((1, N_EXPERT), jnp.float32))

    padded = jnp.ceil(counts / BM) * BM
    er = jax.lax.broadcasted_iota(jnp.int32, (N_EXPERT, N_EXPERT), 0)
    ec = jax.lax.broadcasted_iota(jnp.int32, (N_EXPERT, N_EXPERT), 1)
    etri = (er < ec).astype(jnp.float32)
    pad_off = jnp.dot(padded, etri,
                      preferred_element_type=jnp.float32)

    dest = jnp.sum((rank_s[...] + pad_off) * oh_s[...], axis=-1,
                   keepdims=True)
    o_dest[...] = dest.astype(jnp.int32)

    pad_end = pad_off + padded
    total = jnp.sum(padded).astype(jnp.float32)
    bstart = (jax.lax.broadcasted_iota(jnp.int32, (128, 1), 0)
              .astype(jnp.float32) * BM)
    raw = jnp.sum((pad_end <= bstart).astype(jnp.int32), axis=-1,
                  keepdims=True)
    last_used = jnp.sum((pad_end < total).astype(jnp.int32))
    o_blk[...] = jnp.where(bstart < total, raw, last_used)
    o_nv[...] = jnp.full((8, 1), (total / BM), jnp.float32).astype(jnp.int32)


def _invert_body(dest_ref, tok_ref):
    def _inv(p, _):
        d = dest_ref[p]
        tok_ref[d] = p - jnp.where(p >= T, T, 0)
        return 0
    jax.lax.fori_loop(0, NPAIR, _inv, 0, unroll=8)


def _sc_gather_body(x_hbm, idx_hbm, out_hbm, idx_v, rows_v, sem):
    wid = jax.lax.axis_index("s") * SC_NC + jax.lax.axis_index("c")
    base = wid * SC_ROWS
    for ch in range(SC_NCHUNK):
        off = base + ch * SC_CHUNK
        pltpu.sync_copy(idx_hbm.at[pl.ds(off, SC_CHUNK)], idx_v)
        for j in range(SC_CHUNK // 16):
            v = idx_v[pl.ds(j * 16, 16)]
            idx_v[pl.ds(j * 16, 16)] = jnp.minimum(jnp.maximum(v, 0), T - 1)
        pltpu.async_copy(x_hbm.at[idx_v], rows_v, sem).wait()
        pltpu.sync_copy(rows_v, out_hbm.at[pl.ds(off, SC_CHUNK)])


def _ffn_body(blk_ref, nv_ref, xg_ref, w1_ref, w2_ref, out_ref):
    b = pl.program_id(0)

    @pl.when(b < nv_ref[0])
    def _work():
        xw1 = _dot_t(xg_ref[...], w1_ref[0])
        h = xw1 * jax.nn.sigmoid(xw1)
        out_ref[...] = _dot_t(h, w2_ref[0])


def _combine_body(dest_ref, po_ref, p_ref, y_ref):
    tb = pl.program_id(0)

    def _comb(i, _):
        t = tb * 256 + i
        d1 = dest_ref[t]
        d2 = dest_ref[T + t]
        y_ref[i, :] = (p_ref[t] * po_ref[d1, :]
                       + p_ref[T + t] * po_ref[d2, :])
        return 0
    jax.lax.fori_loop(0, 256, _comb, 0)


@functools.partial(jax.jit, static_argnames=())
def kernel(x, Wg, W1, W2):
    B, Tn, C = x.shape
    x_flat = x.reshape(Tn, C)

    scores = x_flat @ Wg.T

    dest, prob, blk, nv = pl.pallas_call(
        _router_body,
        in_specs=[pl.BlockSpec((T, N_EXPERT), lambda: (0, 0))],
        out_specs=[
            pl.BlockSpec((NPAIR, 1), lambda: (0, 0)),
            pl.BlockSpec((NPAIR, 1), lambda: (0, 0)),
            pl.BlockSpec((128, 1), lambda: (0, 0)),
            pl.BlockSpec((8, 1), lambda: (0, 0)),
        ],
        out_shape=[
            jax.ShapeDtypeStruct((NPAIR, 1), jnp.int32),
            jax.ShapeDtypeStruct((NPAIR, 1), jnp.float32),
            jax.ShapeDtypeStruct((128, 1), jnp.int32),
            jax.ShapeDtypeStruct((8, 1), jnp.int32),
        ],
        scratch_shapes=[
            pltpu.VMEM((NPAIR, N_EXPERT), jnp.float32),
            pltpu.VMEM((NPAIR, N_EXPERT), jnp.float32),
        ],
    )(scores)

    dest_flat = dest.reshape(NPAIR)

    tok_s = pl.pallas_call(
        _invert_body,
        in_specs=[pl.BlockSpec(memory_space=pltpu.SMEM)],
        out_specs=pl.BlockSpec(memory_space=pltpu.SMEM),
        out_shape=jax.ShapeDtypeStruct((NPAD,), jnp.int32),
    )(dest_flat)

    xg = pl.kernel(
        _sc_gather_body,
        mesh=plsc.VectorSubcoreMesh(core_axis_name="c", subcore_axis_name="s"),
        out_type=jax.ShapeDtypeStruct((NPAD, C), jnp.float32),
        scratch_types=[
            pltpu.VMEM((SC_CHUNK,), jnp.int32),
            pltpu.VMEM((SC_CHUNK, C), jnp.float32),
            pltpu.SemaphoreType.DMA,
        ],
    )(x_flat, tok_s)

    pair_out = pl.pallas_call(
        _ffn_body,
        grid_spec=pltpu.PrefetchScalarGridSpec(
            num_scalar_prefetch=2,
            grid=(NB,),
            in_specs=[
                pl.BlockSpec((BM, C), lambda b, blk, nv: (b, 0)),
                pl.BlockSpec((1, FF, C), lambda b, blk, nv: (blk[b], 0, 0)),
                pl.BlockSpec((1, C, FF), lambda b, blk, nv: (blk[b], 0, 0)),
            ],
            out_specs=pl.BlockSpec((BM, C), lambda b, blk, nv: (b, 0)),
        ),
        out_shape=jax.ShapeDtypeStruct((NPAD, C), jnp.float32),
    )(blk.reshape(128), nv.reshape(8)[:1], xg, W1, W2)

    y = pl.pallas_call(
        _combine_body,
        grid_spec=pltpu.PrefetchScalarGridSpec(
            num_scalar_prefetch=1,
            grid=(T // 256,),
            in_specs=[
                pl.BlockSpec((NPAD, C), lambda tb, dest: (0, 0)),
                pl.BlockSpec(memory_space=pltpu.SMEM),
            ],
            out_specs=pl.BlockSpec((256, C), lambda tb, dest: (tb, 0)),
        ),
        out_shape=jax.ShapeDtypeStruct((T, C), jnp.float32),
    )(dest_flat, pair_out, prob.reshape(NPAIR))

    return y.reshape(B, Tn, C)
